# baseline (speedup 1.0000x reference)
import functools

import jax
import jax.numpy as jnp
from jax import lax
from jax.experimental import pallas as pl
from jax.experimental.pallas import tpu as pltpu
from jax.experimental.pallas import tpu_sc as plsc

_NC, _NS, _L = 2, 16, 16
_NW = _NC * _NS
_CH = 32


def _make_sc_kernel(B, S, D):
    spw = S // _NW
    n_chunks = spw // _CH
    chunk = _CH * D

    mesh = plsc.VectorSubcoreMesh(
        core_axis_name="c", subcore_axis_name="s",
        num_cores=_NC, num_subcores=_NS,
    )

    @functools.partial(
        pl.kernel,
        out_type=jax.ShapeDtypeStruct((B * S * D,), jnp.float32),
        mesh=mesh,
        scratch_types=[
            pltpu.VMEM((chunk,), jnp.float32),
            pltpu.VMEM((chunk,), jnp.float32),
        ],
    )
    def sc_add(x_hbm, emb_hbm, out_hbm, emb_v, xbuf):
        wid = lax.axis_index("s") * _NC + lax.axis_index("c")

        def chunk_loop(c, _):
            row0 = (wid * spw + c * _CH) * D
            pltpu.sync_copy(emb_hbm.at[pl.ds(row0, chunk)], emb_v)

            def batch_loop(b, _):
                base = b * S * D + row0
                pltpu.sync_copy(x_hbm.at[pl.ds(base, chunk)], xbuf)

                def add_loop(i, _):
                    off = i * _L
                    plsc.addupdate(
                        xbuf.at[pl.ds(off, _L)], emb_v[pl.ds(off, _L)]
                    )
                    return 0

                lax.fori_loop(0, chunk // _L, add_loop, 0, unroll=8)
                pltpu.sync_copy(xbuf, out_hbm.at[pl.ds(base, chunk)])
                return 0

            return lax.fori_loop(0, B, batch_loop, 0)

        lax.fori_loop(0, n_chunks, chunk_loop, 0)

    return sc_add


def kernel(x, emb_weight):
    B, S, D = x.shape
    sc_add = _make_sc_kernel(B, S, D)
    out = sc_add(x.reshape(-1), emb_weight.reshape(-1))
    return out.reshape(B, S, D)
